# Initial kernel scaffold; baseline (speedup 1.0000x reference)
import jax
import jax.numpy as jnp
from jax.experimental import pallas as pl


def kernel(x, gate_w, w_gate, w_up, w_down):
    raise NotImplementedError("write your pallas kernel here")


# speedup vs baseline: 1.3234x; 1.3234x over previous
import functools

import jax
import jax.numpy as jnp
from jax.experimental import pallas as pl
from jax.experimental.pallas import tpu as pltpu

B, S, DIM = 1, 2048, 768
E = 16
HID = 2688
HB = 384
NHB = HID // HB


def _router_body(x_ref, gw_ref, idx_ref, z_ref):
    x = x_ref[...]
    gw = gw_ref[...]
    logits = jax.lax.dot_general(
        x, gw, (((1,), (1,)), ((), ())), preferred_element_type=jnp.float32
    )
    m = jnp.max(logits, axis=1, keepdims=True)
    lse = m + jnp.log(jnp.sum(jnp.exp(logits - m), axis=1, keepdims=True))
    z_ref[...] = 1e-05 * jnp.sum(lse * lse, axis=0, keepdims=True) / (B * S)
    eq = logits == m
    ii = jax.lax.broadcasted_iota(jnp.int32, logits.shape, 1)
    idx_ref[...] = jnp.min(jnp.where(eq, ii, E), axis=1, keepdims=True)


def _router(x2d, gate_w):
    return pl.pallas_call(
        _router_body,
        out_shape=(
            jax.ShapeDtypeStruct((S, 1), jnp.int32),
            jax.ShapeDtypeStruct((1, 1), jnp.float32),
        ),
    )(x2d, gate_w)


def _ffn_body(x_ref, idx_ref, wg_ref, wu_ref, wd_ref, out_ref):
    e = pl.program_id(0)
    h = pl.program_id(1)

    @pl.when(jnp.logical_and(e == 0, h == 0))
    def _():
        out_ref[...] = jnp.zeros_like(out_ref)

    x = x_ref[...]
    wg = wg_ref[0]
    wu = wu_ref[0]
    wd = wd_ref[0]
    gate = jax.lax.dot_general(
        x, wg, (((1,), (1,)), ((), ())), preferred_element_type=jnp.float32
    )
    up = jax.lax.dot_general(
        x, wu, (((1,), (1,)), ((), ())), preferred_element_type=jnp.float32
    )
    act = gate * (1.0 / (1.0 + jnp.exp(-gate))) * up
    partial = jax.lax.dot_general(
        act, wd, (((1,), (1,)), ((), ())), preferred_element_type=jnp.float32
    )
    tok_w = (idx_ref[...] == e).astype(jnp.float32)
    out_ref[...] += partial * tok_w


def _ffn(x2d, idx, w_gate, w_up, w_down):
    return pl.pallas_call(
        _ffn_body,
        grid=(E, NHB),
        in_specs=[
            pl.BlockSpec((S, DIM), lambda e, h: (0, 0)),
            pl.BlockSpec((S, 1), lambda e, h: (0, 0)),
            pl.BlockSpec((1, HB, DIM), lambda e, h: (e, h, 0)),
            pl.BlockSpec((1, HB, DIM), lambda e, h: (e, h, 0)),
            pl.BlockSpec((1, DIM, HB), lambda e, h: (e, 0, h)),
        ],
        out_specs=pl.BlockSpec((S, DIM), lambda e, h: (0, 0)),
        out_shape=jax.ShapeDtypeStruct((S, DIM), jnp.float32),
    )(x2d, idx, w_gate, w_up, w_down)


def kernel(x, gate_w, w_gate, w_up, w_down):
    x2d = x.reshape(S, DIM)
    idx, z = _router(x2d, gate_w)
    out = _ffn(x2d, idx, w_gate, w_up, w_down)
    expert_indices = idx.reshape(B, S, 1)
    expert_weights = jnp.ones((B, S, 1), jnp.float32)
    return (out.reshape(B, S, DIM), z[0, 0], expert_indices, expert_weights)


# speedup vs baseline: 2.0846x; 1.5753x over previous
import functools

import jax
import jax.numpy as jnp
from jax.experimental import pallas as pl
from jax.experimental.pallas import tpu as pltpu

B, S, DIM = 1, 2048, 768
E = 16
HID = 2688
HB = 384
NHB = HID // HB
TM = 128
NT = S // TM
NI = NT + E - 1


def _router_body(x_ref, gw_ref, idx_ref, z_ref):
    x = x_ref[...]
    gw = gw_ref[...]
    logits = jax.lax.dot_general(
        x, gw, (((1,), (1,)), ((), ())), preferred_element_type=jnp.float32
    )
    m = jnp.max(logits, axis=1, keepdims=True)
    lse = m + jnp.log(jnp.sum(jnp.exp(logits - m), axis=1, keepdims=True))
    z_ref[...] = 1e-05 * jnp.sum(lse * lse, axis=0, keepdims=True) / (B * S)
    eq = logits == m
    ii = jax.lax.broadcasted_iota(jnp.int32, logits.shape, 1)
    idx_ref[...] = jnp.min(jnp.where(eq, ii, E), axis=1, keepdims=True)


def _router(x2d, gate_w):
    return pl.pallas_call(
        _router_body,
        out_shape=(
            jax.ShapeDtypeStruct((S, 1), jnp.int32),
            jax.ShapeDtypeStruct((1, 1), jnp.float32),
        ),
    )(x2d, gate_w)


def _schedule(idx_flat):
    counts = jnp.sum(
        (idx_flat[None, :] == jnp.arange(E, dtype=jnp.int32)[:, None]).astype(
            jnp.int32
        ),
        axis=1,
    )
    ends = jnp.cumsum(counts)
    starts = ends - counts
    t_start = starts // TM
    t_last = jnp.maximum(ends - 1, 0) // TM
    ntiles = jnp.where(counts > 0, t_last - t_start + 1, 0)
    cum_tiles = jnp.cumsum(ntiles)
    sched_start = cum_tiles - ntiles
    total = cum_tiles[-1]
    i_ar = jnp.arange(NI, dtype=jnp.int32)
    gid = jnp.searchsorted(cum_tiles, i_ar, side="right")
    gid = jnp.minimum(gid, E - 1).astype(jnp.int32)
    valid = i_ar < total
    mt = t_start[gid] + (i_ar - sched_start[gid])
    mt = jnp.where(valid, mt, NT - 1).astype(jnp.int32)
    lo = jnp.where(valid, jnp.maximum(starts[gid], mt * TM), 0).astype(jnp.int32)
    hi = jnp.where(valid, jnp.minimum(ends[gid], (mt + 1) * TM), 0).astype(jnp.int32)
    gid = jnp.where(valid, gid, E - 1).astype(jnp.int32)
    perm = jnp.argsort(idx_flat, stable=True).astype(jnp.int32)
    return perm, gid, mt, lo, hi


def _gmm_body(gid_ref, mt_ref, lo_ref, hi_ref, x_ref, wg_ref, wu_ref, wd_ref, out_ref):
    h = pl.program_id(0)
    i = pl.program_id(1)

    @pl.when(jnp.logical_and(h == 0, i == 0))
    def _():
        out_ref[...] = jnp.zeros_like(out_ref)

    x = x_ref[...]
    wg = wg_ref[0]
    wu = wu_ref[0]
    wd = wd_ref[0]
    gate = jax.lax.dot_general(
        x, wg, (((1,), (1,)), ((), ())), preferred_element_type=jnp.float32
    )
    up = jax.lax.dot_general(
        x, wu, (((1,), (1,)), ((), ())), preferred_element_type=jnp.float32
    )
    act = gate * (1.0 / (1.0 + jnp.exp(-gate))) * up
    partial = jax.lax.dot_general(
        act, wd, (((1,), (1,)), ((), ())), preferred_element_type=jnp.float32
    )
    row0 = mt_ref[i] * TM
    rows = row0 + jax.lax.broadcasted_iota(jnp.int32, (TM, 1), 0)
    m = ((rows >= lo_ref[i]) & (rows < hi_ref[i])).astype(jnp.float32)
    out_ref[pl.ds(pl.multiple_of(row0, TM), TM), :] += m * partial


def _gmm(x_sorted, gid, mt, lo, hi, w_gate, w_up, w_down):
    grid_spec = pltpu.PrefetchScalarGridSpec(
        num_scalar_prefetch=4,
        grid=(NHB, NI),
        in_specs=[
            pl.BlockSpec((TM, DIM), lambda h, i, g, t, l, u: (t[i], 0)),
            pl.BlockSpec((1, HB, DIM), lambda h, i, g, t, l, u: (g[i], h, 0)),
            pl.BlockSpec((1, HB, DIM), lambda h, i, g, t, l, u: (g[i], h, 0)),
            pl.BlockSpec((1, DIM, HB), lambda h, i, g, t, l, u: (g[i], 0, h)),
        ],
        out_specs=pl.BlockSpec((S, DIM), lambda h, i, g, t, l, u: (0, 0)),
    )
    return pl.pallas_call(
        _gmm_body,
        grid_spec=grid_spec,
        out_shape=jax.ShapeDtypeStruct((S, DIM), jnp.float32),
    )(gid, mt, lo, hi, x_sorted, w_gate, w_up, w_down)


def kernel(x, gate_w, w_gate, w_up, w_down):
    x2d = x.reshape(S, DIM)
    idx, z = _router(x2d, gate_w)
    perm, gid, mt, lo, hi = _schedule(idx[:, 0])
    x_sorted = jnp.take(x2d, perm, axis=0)
    out_sorted = _gmm(x_sorted, gid, mt, lo, hi, w_gate, w_up, w_down)
    out = jnp.zeros((S, DIM), jnp.float32).at[perm].set(out_sorted)
    expert_indices = idx.reshape(B, S, 1)
    expert_weights = jnp.ones((B, S, 1), jnp.float32)
    return (out.reshape(B, S, DIM), z[0, 0], expert_indices, expert_weights)


# speedup vs baseline: 2.6631x; 1.2775x over previous
import functools

import jax
import jax.numpy as jnp
from jax.experimental import pallas as pl
from jax.experimental.pallas import tpu as pltpu

B, S, DIM = 1, 2048, 768
E = 16
HID = 2688
HB = 896
NHB = HID // HB
TM = 128
NT = S // TM
NI = NT + E - 1


def _router_body(x_ref, gw_ref, idx_ref, z_ref):
    x = x_ref[...]
    gw = gw_ref[...]
    logits = jax.lax.dot_general(
        x, gw, (((1,), (1,)), ((), ())), preferred_element_type=jnp.float32
    )
    m = jnp.max(logits, axis=1, keepdims=True)
    lse = m + jnp.log(jnp.sum(jnp.exp(logits - m), axis=1, keepdims=True))
    z_ref[...] = 1e-05 * jnp.sum(lse * lse, axis=0, keepdims=True) / (B * S)
    eq = logits == m
    ii = jax.lax.broadcasted_iota(jnp.int32, logits.shape, 1)
    idx_ref[...] = jnp.min(jnp.where(eq, ii, E), axis=1, keepdims=True)


def _router(x2d, gate_w):
    return pl.pallas_call(
        _router_body,
        out_shape=(
            jax.ShapeDtypeStruct((S, 1), jnp.int32),
            jax.ShapeDtypeStruct((1, 1), jnp.float32),
        ),
    )(x2d, gate_w)


def _schedule(idx_flat):
    counts = jnp.sum(
        (idx_flat[None, :] == jnp.arange(E, dtype=jnp.int32)[:, None]).astype(
            jnp.int32
        ),
        axis=1,
    )
    ends = jnp.cumsum(counts)
    starts = ends - counts
    t_start = starts // TM
    t_last = jnp.maximum(ends - 1, 0) // TM
    ntiles = jnp.where(counts > 0, t_last - t_start + 1, 0)
    cum_tiles = jnp.cumsum(ntiles)
    sched_start = cum_tiles - ntiles
    total = cum_tiles[-1]
    i_ar = jnp.arange(NI, dtype=jnp.int32)
    gid = jnp.searchsorted(cum_tiles, i_ar, side="right")
    gid = jnp.minimum(gid, E - 1).astype(jnp.int32)
    valid = i_ar < total
    mt = t_start[gid] + (i_ar - sched_start[gid])
    mt = jnp.where(valid, mt, NT - 1).astype(jnp.int32)
    lo = jnp.where(valid, jnp.maximum(starts[gid], mt * TM), 0).astype(jnp.int32)
    hi = jnp.where(valid, jnp.minimum(ends[gid], (mt + 1) * TM), 0).astype(jnp.int32)
    gid = jnp.where(valid, gid, E - 1).astype(jnp.int32)
    perm = jnp.argsort(idx_flat, stable=True).astype(jnp.int32)
    return perm, gid, mt, lo, hi


def _gmm_body(gid_ref, mt_ref, lo_ref, hi_ref, x_ref, wg_ref, wu_ref, wd_ref, out_ref):
    h = pl.program_id(0)
    i = pl.program_id(1)

    @pl.when(jnp.logical_and(h == 0, i == 0))
    def _():
        out_ref[...] = jnp.zeros_like(out_ref)

    row0 = mt_ref[i] * TM
    x = x_ref[pl.ds(pl.multiple_of(row0, TM), TM), :]
    wg = wg_ref[0]
    wu = wu_ref[0]
    wd = wd_ref[0]
    gate = jax.lax.dot_general(
        x, wg, (((1,), (1,)), ((), ())), preferred_element_type=jnp.float32
    )
    up = jax.lax.dot_general(
        x, wu, (((1,), (1,)), ((), ())), preferred_element_type=jnp.float32
    )
    act = gate * (1.0 / (1.0 + jnp.exp(-gate))) * up
    partial = jax.lax.dot_general(
        act, wd, (((1,), (1,)), ((), ())), preferred_element_type=jnp.float32
    )
    rows = row0 + jax.lax.broadcasted_iota(jnp.int32, (TM, 1), 0)
    m = ((rows >= lo_ref[i]) & (rows < hi_ref[i])).astype(jnp.float32)
    out_ref[pl.ds(pl.multiple_of(row0, TM), TM), :] += m * partial


def _gmm(x_sorted, gid, mt, lo, hi, w_gate, w_up, w_down):
    grid_spec = pltpu.PrefetchScalarGridSpec(
        num_scalar_prefetch=4,
        grid=(NHB, NI),
        in_specs=[
            pl.BlockSpec((S, DIM), lambda h, i, g, t, l, u: (0, 0)),
            pl.BlockSpec((1, HB, DIM), lambda h, i, g, t, l, u: (g[i], h, 0)),
            pl.BlockSpec((1, HB, DIM), lambda h, i, g, t, l, u: (g[i], h, 0)),
            pl.BlockSpec((1, DIM, HB), lambda h, i, g, t, l, u: (g[i], 0, h)),
        ],
        out_specs=pl.BlockSpec((S, DIM), lambda h, i, g, t, l, u: (0, 0)),
    )
    return pl.pallas_call(
        _gmm_body,
        grid_spec=grid_spec,
        out_shape=jax.ShapeDtypeStruct((S, DIM), jnp.float32),
    )(gid, mt, lo, hi, x_sorted, w_gate, w_up, w_down)


def kernel(x, gate_w, w_gate, w_up, w_down):
    x2d = x.reshape(S, DIM)
    idx, z = _router(x2d, gate_w)
    perm, gid, mt, lo, hi = _schedule(idx[:, 0])
    x_sorted = jnp.take(x2d, perm, axis=0)
    out_sorted = _gmm(x_sorted, gid, mt, lo, hi, w_gate, w_up, w_down)
    out = jnp.zeros((S, DIM), jnp.float32).at[perm].set(out_sorted)
    expert_indices = idx.reshape(B, S, 1)
    expert_weights = jnp.ones((B, S, 1), jnp.float32)
    return (out.reshape(B, S, DIM), z[0, 0], expert_indices, expert_weights)


# speedup vs baseline: 2.7600x; 1.0364x over previous
import functools

import jax
import jax.numpy as jnp
from jax import lax
from jax.experimental import pallas as pl
from jax.experimental.pallas import tpu as pltpu
from jax.experimental.pallas import tpu_sc as plsc

B, S, DIM = 1, 2048, 768
E = 16
HID = 2688
HB = 896
NHB = HID // HB
TM = 128
NT = S // TM
NI = NT + E - 1


def _router_body(x_ref, gw_ref, idx_ref, z_ref):
    x = x_ref[...]
    gw = gw_ref[...]
    logits = jax.lax.dot_general(
        x, gw, (((1,), (1,)), ((), ())), preferred_element_type=jnp.float32
    )
    m = jnp.max(logits, axis=1, keepdims=True)
    lse = m + jnp.log(jnp.sum(jnp.exp(logits - m), axis=1, keepdims=True))
    z_ref[...] = 1e-05 * jnp.sum(lse * lse, axis=0, keepdims=True) / (B * S)
    eq = logits == m
    ii = jax.lax.broadcasted_iota(jnp.int32, logits.shape, 1)
    idx_ref[...] = jnp.min(jnp.where(eq, ii, E), axis=1, keepdims=True)


def _router(x2d, gate_w):
    return pl.pallas_call(
        _router_body,
        out_shape=(
            jax.ShapeDtypeStruct((S, 1), jnp.int32),
            jax.ShapeDtypeStruct((1, 1), jnp.float32),
        ),
    )(x2d, gate_w)


NC = 2
NS = 16
NW = NC * NS
TPW = S // NW
_SC_MESH = plsc.VectorSubcoreMesh(core_axis_name="c", subcore_axis_name="s")


def _vgather(v, idx):
    dn = lax.GatherDimensionNumbers(
        offset_dims=(), collapsed_slice_dims=(0,), start_index_map=(0,)
    )
    return lax.gather(
        v, idx[:, None], dn, slice_sizes=(1,),
        mode=lax.GatherScatterMode.PROMISE_IN_BOUNDS,
    )


def _vcumsum(v):
    iota16 = lax.iota(jnp.int32, 16)
    one = jnp.full((16,), 1, jnp.int32)
    zero = jnp.zeros((16,), jnp.int32)
    for sh in (1, 2, 4, 8):
        idxs = jnp.maximum(iota16 - sh, zero)
        keep = jnp.minimum(jnp.maximum(iota16 - (sh - 1), zero), one)
        v = v + keep * _vgather(v, idxs)
    return v


def _lane_bcast(v, lane):
    return _vgather(v, jnp.full((16,), lane, jnp.int32))


def _eq(a, b):
    one = jnp.full((16,), 1, jnp.int32)
    return one - jnp.minimum(jnp.abs(a - b), one)


def _dispatch_body(
    idx_hbm, x_hbm, hist_hbm, spos_hbm, xs_hbm, idx_v, hrow_v, allh_v, pos_v, xrows_v, sem
):
    wid = lax.axis_index("s") * NC + lax.axis_index("c")
    base = wid * TPW
    pltpu.sync_copy(idx_hbm.at[pl.ds(base, TPW)], idx_v)
    iota16 = lax.iota(jnp.int32, 16)
    zeros16 = jnp.zeros((16,), jnp.int32)
    ones16 = jnp.full((16,), 1, jnp.int32)
    widv = lax.broadcast(wid, (16,))
    hist = zeros16
    for b in range(E):
        bv = jnp.full((16,), b, jnp.int32)
        cnt = zeros16
        for k in range(TPW // 16):
            vals = idx_v[pl.ds(k * 16, 16)]
            cnt = cnt + _eq(vals, bv)
        total = _lane_bcast(_vcumsum(cnt), 15)
        hist = hist + _eq(iota16, bv) * total
    hrow_v[...] = hist
    pltpu.sync_copy(hrow_v, hist_hbm.at[wid])
    plsc.subcore_barrier()
    pltpu.sync_copy(hist_hbm, allh_v)
    totals = zeros16
    prev = zeros16
    for w in range(NW):
        row = allh_v[w]
        totals = totals + row
        wsel = jnp.minimum(jnp.maximum(widv - w, zeros16), ones16)
        prev = prev + wsel * row
    bin_excl = _vcumsum(totals) - totals
    basev = bin_excl + prev
    for k in range(TPW // 16):
        vals = idx_v[pl.ds(k * 16, 16)]
        baseg = _vgather(basev, vals)
        rank = zeros16
        cnts = zeros16
        for b in range(E):
            bv = jnp.full((16,), b, jnp.int32)
            mi = _eq(vals, bv)
            cs = _vcumsum(mi)
            rank = rank + mi * (cs - ones16)
            cnts = cnts + _eq(iota16, bv) * _lane_bcast(cs, 15)
        pos_v[pl.ds(k * 16, 16)] = baseg + rank
        basev = basev + cnts
    pltpu.sync_copy(pos_v, spos_hbm.at[pl.ds(base, TPW)])
    pltpu.sync_copy(x_hbm.at[pl.ds(base, TPW)], xrows_v)
    pltpu.async_copy(xrows_v, xs_hbm.at[pos_v], sem).wait()


def _dispatch(idx_flat, x2d):
    f = functools.partial(
        pl.kernel,
        mesh=_SC_MESH,
        out_type=(
            jax.ShapeDtypeStruct((NW, 16), jnp.int32),
            jax.ShapeDtypeStruct((S,), jnp.int32),
            jax.ShapeDtypeStruct((S, DIM), jnp.float32),
        ),
        scratch_types=[
            pltpu.VMEM((TPW,), jnp.int32),
            pltpu.VMEM((16,), jnp.int32),
            pltpu.VMEM((NW, 16), jnp.int32),
            pltpu.VMEM((TPW,), jnp.int32),
            pltpu.VMEM((TPW, DIM), jnp.float32),
            pltpu.SemaphoreType.DMA,
        ],
    )
    return f(_dispatch_body)(idx_flat, x2d)


def _unsort_body(y_hbm, spos_hbm, out_hbm, pos_v, rows_v, sem):
    wid = lax.axis_index("s") * NC + lax.axis_index("c")
    base = wid * TPW
    pltpu.sync_copy(spos_hbm.at[pl.ds(base, TPW)], pos_v)
    pltpu.async_copy(y_hbm.at[pos_v], rows_v, sem).wait()
    pltpu.sync_copy(rows_v, out_hbm.at[pl.ds(base, TPW)])


def _unsort(y_sorted, spos):
    f = functools.partial(
        pl.kernel,
        mesh=_SC_MESH,
        out_type=jax.ShapeDtypeStruct((S, DIM), jnp.float32),
        scratch_types=[
            pltpu.VMEM((TPW,), jnp.int32),
            pltpu.VMEM((TPW, DIM), jnp.float32),
            pltpu.SemaphoreType.DMA,
        ],
    )
    return f(_unsort_body)(y_sorted, spos)


def _schedule(counts):
    ends = jnp.cumsum(counts)
    starts = ends - counts
    t_start = starts // TM
    t_last = jnp.maximum(ends - 1, 0) // TM
    ntiles = jnp.where(counts > 0, t_last - t_start + 1, 0)
    cum_tiles = jnp.cumsum(ntiles)
    sched_start = cum_tiles - ntiles
    total = cum_tiles[-1]
    i_ar = jnp.arange(NI, dtype=jnp.int32)
    gid = jnp.searchsorted(cum_tiles, i_ar, side="right")
    gid = jnp.minimum(gid, E - 1).astype(jnp.int32)
    valid = i_ar < total
    mt = t_start[gid] + (i_ar - sched_start[gid])
    mt = jnp.where(valid, mt, NT - 1).astype(jnp.int32)
    lo = jnp.where(valid, jnp.maximum(starts[gid], mt * TM), 0).astype(jnp.int32)
    hi = jnp.where(valid, jnp.minimum(ends[gid], (mt + 1) * TM), 0).astype(jnp.int32)
    gid = jnp.where(valid, gid, E - 1).astype(jnp.int32)
    return gid, mt, lo, hi


def _gmm_body(gid_ref, mt_ref, lo_ref, hi_ref, x_ref, wg_ref, wu_ref, wd_ref, out_ref):
    h = pl.program_id(0)
    i = pl.program_id(1)

    @pl.when(jnp.logical_and(h == 0, i == 0))
    def _():
        out_ref[...] = jnp.zeros_like(out_ref)

    row0 = mt_ref[i] * TM
    x = x_ref[pl.ds(pl.multiple_of(row0, TM), TM), :]
    wg = wg_ref[0]
    wu = wu_ref[0]
    wd = wd_ref[0]
    gate = jax.lax.dot_general(
        x, wg, (((1,), (1,)), ((), ())), preferred_element_type=jnp.float32
    )
    up = jax.lax.dot_general(
        x, wu, (((1,), (1,)), ((), ())), preferred_element_type=jnp.float32
    )
    act = gate * (1.0 / (1.0 + jnp.exp(-gate))) * up
    partial = jax.lax.dot_general(
        act, wd, (((1,), (1,)), ((), ())), preferred_element_type=jnp.float32
    )
    rows = row0 + jax.lax.broadcasted_iota(jnp.int32, (TM, 1), 0)
    m = ((rows >= lo_ref[i]) & (rows < hi_ref[i])).astype(jnp.float32)
    out_ref[pl.ds(pl.multiple_of(row0, TM), TM), :] += m * partial


def _gmm(x_sorted, gid, mt, lo, hi, w_gate, w_up, w_down):
    grid_spec = pltpu.PrefetchScalarGridSpec(
        num_scalar_prefetch=4,
        grid=(NHB, NI),
        in_specs=[
            pl.BlockSpec((S, DIM), lambda h, i, g, t, l, u: (0, 0)),
            pl.BlockSpec((1, HB, DIM), lambda h, i, g, t, l, u: (g[i], h, 0)),
            pl.BlockSpec((1, HB, DIM), lambda h, i, g, t, l, u: (g[i], h, 0)),
            pl.BlockSpec((1, DIM, HB), lambda h, i, g, t, l, u: (g[i], 0, h)),
        ],
        out_specs=pl.BlockSpec((S, DIM), lambda h, i, g, t, l, u: (0, 0)),
    )
    return pl.pallas_call(
        _gmm_body,
        grid_spec=grid_spec,
        out_shape=jax.ShapeDtypeStruct((S, DIM), jnp.float32),
    )(gid, mt, lo, hi, x_sorted, w_gate, w_up, w_down)


def kernel(x, gate_w, w_gate, w_up, w_down):
    x2d = x.reshape(S, DIM)
    idx, z = _router(x2d, gate_w)
    hist, spos, x_sorted = _dispatch(idx[:, 0], x2d)
    counts = jnp.sum(hist, axis=0)
    gid, mt, lo, hi = _schedule(counts)
    out_sorted = _gmm(x_sorted, gid, mt, lo, hi, w_gate, w_up, w_down)
    out = _unsort(out_sorted, spos)
    expert_indices = idx.reshape(B, S, 1)
    expert_weights = jnp.ones((B, S, 1), jnp.float32)
    return (out.reshape(B, S, DIM), z[0, 0], expert_indices, expert_weights)


# speedup vs baseline: 3.4606x; 1.2538x over previous
import functools

import jax
import jax.numpy as jnp
from jax import lax
from jax.experimental import pallas as pl
from jax.experimental.pallas import tpu as pltpu
from jax.experimental.pallas import tpu_sc as plsc

B, S, DIM = 1, 2048, 768
E = 16
HID = 2688
HB = 896
NHB = HID // HB
TM = 256
NT = S // TM
NI = NT + E - 1


def _router_body(x_ref, gw_ref, idx_ref, z_ref):
    x = x_ref[...]
    gw = gw_ref[...]
    logits = jax.lax.dot_general(
        x, gw, (((1,), (1,)), ((), ())), preferred_element_type=jnp.float32
    )
    m = jnp.max(logits, axis=1, keepdims=True)
    lse = m + jnp.log(jnp.sum(jnp.exp(logits - m), axis=1, keepdims=True))
    z_ref[...] = 1e-05 * jnp.sum(lse * lse, axis=0, keepdims=True) / (B * S)
    eq = logits == m
    ii = jax.lax.broadcasted_iota(jnp.int32, logits.shape, 1)
    idx_ref[...] = jnp.min(jnp.where(eq, ii, E), axis=1, keepdims=True)


def _router(x2d, gate_w):
    return pl.pallas_call(
        _router_body,
        out_shape=(
            jax.ShapeDtypeStruct((S, 1), jnp.int32),
            jax.ShapeDtypeStruct((1, 1), jnp.float32),
        ),
    )(x2d, gate_w)


NC = 2
NS = 16
NW = NC * NS
TPW = S // NW
_SC_MESH = plsc.VectorSubcoreMesh(core_axis_name="c", subcore_axis_name="s")


def _vgather(v, idx):
    dn = lax.GatherDimensionNumbers(
        offset_dims=(), collapsed_slice_dims=(0,), start_index_map=(0,)
    )
    return lax.gather(
        v, idx[:, None], dn, slice_sizes=(1,),
        mode=lax.GatherScatterMode.PROMISE_IN_BOUNDS,
    )


def _vcumsum(v):
    iota16 = lax.iota(jnp.int32, 16)
    one = jnp.full((16,), 1, jnp.int32)
    zero = jnp.zeros((16,), jnp.int32)
    for sh in (1, 2, 4, 8):
        idxs = jnp.maximum(iota16 - sh, zero)
        keep = jnp.minimum(jnp.maximum(iota16 - (sh - 1), zero), one)
        v = v + keep * _vgather(v, idxs)
    return v


def _lane_bcast(v, lane):
    return _vgather(v, jnp.full((16,), lane, jnp.int32))


def _eq(a, b):
    one = jnp.full((16,), 1, jnp.int32)
    return one - jnp.minimum(jnp.abs(a - b), one)


def _dispatch_body(
    idx_hbm, x_hbm, hist_hbm, spos_hbm, xs_hbm, idx_v, hrow_v, allh_v, pos_v, xrows_v, sem
):
    wid = lax.axis_index("s") * NC + lax.axis_index("c")
    base = wid * TPW
    pltpu.sync_copy(idx_hbm.at[pl.ds(base, TPW)], idx_v)
    iota16 = lax.iota(jnp.int32, 16)
    zeros16 = jnp.zeros((16,), jnp.int32)
    ones16 = jnp.full((16,), 1, jnp.int32)
    widv = lax.broadcast(wid, (16,))
    hist = zeros16
    for b in range(E):
        bv = jnp.full((16,), b, jnp.int32)
        cnt = zeros16
        for k in range(TPW // 16):
            vals = idx_v[pl.ds(k * 16, 16)]
            cnt = cnt + _eq(vals, bv)
        total = _lane_bcast(_vcumsum(cnt), 15)
        hist = hist + _eq(iota16, bv) * total
    hrow_v[...] = hist
    pltpu.sync_copy(hrow_v, hist_hbm.at[wid])
    plsc.subcore_barrier()
    pltpu.sync_copy(hist_hbm, allh_v)
    totals = zeros16
    prev = zeros16
    for w in range(NW):
        row = allh_v[w]
        totals = totals + row
        wsel = jnp.minimum(jnp.maximum(widv - w, zeros16), ones16)
        prev = prev + wsel * row
    bin_excl = _vcumsum(totals) - totals
    basev = bin_excl + prev
    for k in range(TPW // 16):
        vals = idx_v[pl.ds(k * 16, 16)]
        baseg = _vgather(basev, vals)
        rank = zeros16
        cnts = zeros16
        for b in range(E):
            bv = jnp.full((16,), b, jnp.int32)
            mi = _eq(vals, bv)
            cs = _vcumsum(mi)
            rank = rank + mi * (cs - ones16)
            cnts = cnts + _eq(iota16, bv) * _lane_bcast(cs, 15)
        pos_v[pl.ds(k * 16, 16)] = baseg + rank
        basev = basev + cnts
    pltpu.sync_copy(pos_v, spos_hbm.at[pl.ds(base, TPW)])
    pltpu.sync_copy(x_hbm.at[pl.ds(base, TPW)], xrows_v)
    pltpu.async_copy(xrows_v, xs_hbm.at[pos_v], sem).wait()


def _dispatch(idx_flat, x2d):
    f = functools.partial(
        pl.kernel,
        mesh=_SC_MESH,
        out_type=(
            jax.ShapeDtypeStruct((NW, 16), jnp.int32),
            jax.ShapeDtypeStruct((S,), jnp.int32),
            jax.ShapeDtypeStruct((S, DIM), jnp.float32),
        ),
        scratch_types=[
            pltpu.VMEM((TPW,), jnp.int32),
            pltpu.VMEM((16,), jnp.int32),
            pltpu.VMEM((NW, 16), jnp.int32),
            pltpu.VMEM((TPW,), jnp.int32),
            pltpu.VMEM((TPW, DIM), jnp.float32),
            pltpu.SemaphoreType.DMA,
        ],
    )
    return f(_dispatch_body)(idx_flat, x2d)


def _unsort_body(y_hbm, spos_hbm, out_hbm, pos_v, rows_v, sem):
    wid = lax.axis_index("s") * NC + lax.axis_index("c")
    base = wid * TPW
    pltpu.sync_copy(spos_hbm.at[pl.ds(base, TPW)], pos_v)
    pltpu.async_copy(y_hbm.at[pos_v], rows_v, sem).wait()
    pltpu.sync_copy(rows_v, out_hbm.at[pl.ds(base, TPW)])


def _unsort(y_sorted, spos):
    f = functools.partial(
        pl.kernel,
        mesh=_SC_MESH,
        out_type=jax.ShapeDtypeStruct((S, DIM), jnp.float32),
        scratch_types=[
            pltpu.VMEM((TPW,), jnp.int32),
            pltpu.VMEM((TPW, DIM), jnp.float32),
            pltpu.SemaphoreType.DMA,
        ],
    )
    return f(_unsort_body)(y_sorted, spos)


def _schedule(counts):
    ends = jnp.cumsum(counts)
    starts = ends - counts
    t_start = starts // TM
    t_last = jnp.maximum(ends - 1, 0) // TM
    ntiles = jnp.where(counts > 0, t_last - t_start + 1, 0)
    cum_tiles = jnp.cumsum(ntiles)
    sched_start = cum_tiles - ntiles
    total = cum_tiles[-1]
    i_ar = jnp.arange(NI, dtype=jnp.int32)
    gid = jnp.searchsorted(cum_tiles, i_ar, side="right")
    gid = jnp.minimum(gid, E - 1).astype(jnp.int32)
    valid = i_ar < total
    mt = t_start[gid] + (i_ar - sched_start[gid])
    mt = jnp.where(valid, mt, NT - 1).astype(jnp.int32)
    lo = jnp.where(valid, jnp.maximum(starts[gid], mt * TM), 0).astype(jnp.int32)
    hi = jnp.where(valid, jnp.minimum(ends[gid], (mt + 1) * TM), 0).astype(jnp.int32)
    gid = jnp.where(valid, gid, E - 1).astype(jnp.int32)
    return gid, mt, lo, hi


def _gmm_body(gid_ref, mt_ref, lo_ref, hi_ref, x_ref, wg_ref, wu_ref, wd_ref, out_ref):
    h = pl.program_id(0)
    i = pl.program_id(1)

    @pl.when(jnp.logical_and(h == 0, i == 0))
    def _():
        out_ref[...] = jnp.zeros_like(out_ref)

    row0 = mt_ref[i] * TM
    x = x_ref[pl.ds(pl.multiple_of(row0, TM), TM), :]
    wg = wg_ref[0]
    wu = wu_ref[0]
    wd = wd_ref[0]
    gate = jax.lax.dot_general(
        x, wg, (((1,), (1,)), ((), ())), preferred_element_type=jnp.float32
    )
    up = jax.lax.dot_general(
        x, wu, (((1,), (1,)), ((), ())), preferred_element_type=jnp.float32
    )
    act = gate * (1.0 / (1.0 + jnp.exp(-gate))) * up
    partial = jax.lax.dot_general(
        act, wd, (((1,), (1,)), ((), ())), preferred_element_type=jnp.float32
    )
    rows = row0 + jax.lax.broadcasted_iota(jnp.int32, (TM, 1), 0)
    m = ((rows >= lo_ref[i]) & (rows < hi_ref[i])).astype(jnp.float32)
    out_ref[pl.ds(pl.multiple_of(row0, TM), TM), :] += m * partial


def _gmm(x_sorted, gid, mt, lo, hi, w_gate, w_up, w_down):
    grid_spec = pltpu.PrefetchScalarGridSpec(
        num_scalar_prefetch=4,
        grid=(NHB, NI),
        in_specs=[
            pl.BlockSpec((S, DIM), lambda h, i, g, t, l, u: (0, 0)),
            pl.BlockSpec((1, HB, DIM), lambda h, i, g, t, l, u: (g[i], h, 0)),
            pl.BlockSpec((1, HB, DIM), lambda h, i, g, t, l, u: (g[i], h, 0)),
            pl.BlockSpec((1, DIM, HB), lambda h, i, g, t, l, u: (g[i], 0, h)),
        ],
        out_specs=pl.BlockSpec((S, DIM), lambda h, i, g, t, l, u: (0, 0)),
    )
    return pl.pallas_call(
        _gmm_body,
        grid_spec=grid_spec,
        out_shape=jax.ShapeDtypeStruct((S, DIM), jnp.float32),
    )(gid, mt, lo, hi, x_sorted, w_gate, w_up, w_down)


def kernel(x, gate_w, w_gate, w_up, w_down):
    x2d = x.reshape(S, DIM)
    idx, z = _router(x2d, gate_w)
    hist, spos, x_sorted = _dispatch(idx[:, 0], x2d)
    counts = jnp.sum(hist, axis=0)
    gid, mt, lo, hi = _schedule(counts)
    out_sorted = _gmm(x_sorted, gid, mt, lo, hi, w_gate, w_up, w_down)
    out = _unsort(out_sorted, spos)
    expert_indices = idx.reshape(B, S, 1)
    expert_weights = jnp.ones((B, S, 1), jnp.float32)
    return (out.reshape(B, S, DIM), z[0, 0], expert_indices, expert_weights)
